# baseline (speedup 1.0000x reference)
import functools

import jax
import jax.numpy as jnp
from jax import lax
from jax.experimental import pallas as pl
from jax.experimental.pallas import tpu as pltpu
from jax.experimental.pallas import tpu_sc as plsc

D = 64
B = 4096
MARGIN = 1.0

_NC = 2
_NS = 16
_NW = _NC * _NS
_JOBS = 2 * B
_JPW = _JOBS // _NW
_CH = 16
_NCHUNK = _JPW // _CH


def _allsum16(v):
    idx = lax.iota(jnp.int32, 16)
    for sh in (1, 2, 4, 8):
        v = v + v.at[idx ^ sh].get(mode="promise_in_bounds")
    return v


def _rsqrt16(x):
    i = lax.bitcast_convert_type(x, jnp.int32)
    y = lax.bitcast_convert_type(jnp.int32(0x5F3759DF) - (i >> 1), jnp.float32)
    for _ in range(3):
        y = y * (1.5 - 0.5 * x * y * y)
    return y


def _sc_body(hid_hbm, tid_hbm, rid_hbm, ent_hbm, rel_hbm, tr_hbm, out_hbm,
             hid_v, tid_v, rid_v, m_v, h_v, t_v, r_v, acc_v, sem):
    wid = lax.axis_index("s") * _NC + lax.axis_index("c")
    base = wid * _JPW
    pltpu.sync_copy(hid_hbm.at[pl.ds(base, _JPW)], hid_v)
    pltpu.sync_copy(tid_hbm.at[pl.ds(base, _JPW)], tid_v)
    pltpu.sync_copy(rid_hbm.at[pl.ds(base, _JPW)], rid_v)

    def job_scores(j):
        ah = [jnp.zeros((16,), jnp.float32) for _ in range(4)]
        at = [jnp.zeros((16,), jnp.float32) for _ in range(4)]

        def dstep(d16, carry):
            ah0, ah1, ah2, ah3, at0, at1, at2, at3 = carry
            hv = h_v[j, pl.ds(d16 * 16, 16)]
            tv = t_v[j, pl.ds(d16 * 16, 16)]
            for dd in range(16):
                he = hv[dd]
                te = tv[dd]
                off = (d16 * 16 + dd) * D
                m0 = m_v[j, pl.ds(off, 16)]
                m1 = m_v[j, pl.ds(off + 16, 16)]
                m2 = m_v[j, pl.ds(off + 32, 16)]
                m3 = m_v[j, pl.ds(off + 48, 16)]
                ah0 = ah0 + he * m0
                ah1 = ah1 + he * m1
                ah2 = ah2 + he * m2
                ah3 = ah3 + he * m3
                at0 = at0 + te * m0
                at1 = at1 + te * m1
                at2 = at2 + te * m2
                at3 = at3 + te * m3
            return (ah0, ah1, ah2, ah3, at0, at1, at2, at3)

        ah[0], ah[1], ah[2], ah[3], at[0], at[1], at[2], at[3] = lax.fori_loop(
            0, D // 16, dstep, (*ah, *at))

        ssh = ah[0] * ah[0] + ah[1] * ah[1] + ah[2] * ah[2] + ah[3] * ah[3]
        sst = at[0] * at[0] + at[1] * at[1] + at[2] * at[2] + at[3] * at[3]
        inv_h = _rsqrt16(jnp.maximum(_allsum16(ssh), 1e-12))
        inv_t = _rsqrt16(jnp.maximum(_allsum16(sst), 1e-12))

        s_abs = jnp.zeros((16,), jnp.float32)
        s_sq = jnp.zeros((16,), jnp.float32)
        for k in range(4):
            dk = inv_h * ah[k] + r_v[j, pl.ds(16 * k, 16)] - inv_t * at[k]
            s_abs = s_abs + jnp.abs(dk)
            s_sq = s_sq + dk * dk
        return _allsum16(s_abs), _allsum16(s_sq)

    def chunk_body(c, carry):
        acc1, acc2 = carry
        cb = c * _CH
        hidx = hid_v[pl.ds(cb, _CH)]
        tidx = tid_v[pl.ds(cb, _CH)]
        ridx = rid_v[pl.ds(cb, _CH)]
        cp_m = pltpu.async_copy(tr_hbm.at[ridx], m_v, sem)
        cp_h = pltpu.async_copy(ent_hbm.at[hidx], h_v, sem)
        cp_t = pltpu.async_copy(ent_hbm.at[tidx], t_v, sem)
        cp_r = pltpu.async_copy(rel_hbm.at[ridx], r_v, sem)
        cp_m.wait()
        cp_h.wait()
        cp_t.wait()
        cp_r.wait()
        for p in range(_CH // 2):
            p1, p2 = job_scores(2 * p)
            n1, n2 = job_scores(2 * p + 1)
            acc1 = acc1 + jnp.maximum(p1 - n1 + MARGIN, 0.0)
            acc2 = acc2 + jnp.maximum(p2 - n2 + MARGIN, 0.0)
        return acc1, acc2

    zero16 = jnp.zeros((16,), jnp.float32)
    acc1, acc2 = lax.fori_loop(0, _NCHUNK, chunk_body, (zero16, zero16))

    lane = lax.iota(jnp.int32, 16)
    res = jnp.where(lane == 0, acc1, jnp.where(lane == 1, acc2, 0.0))
    acc_v[...] = res
    pltpu.sync_copy(acc_v, out_hbm.at[wid])


def _run_sc(x, ent_emb, rel_emb, transfer):
    pos_h, pos_t, pos_r = x[:, 0], x[:, 1], x[:, 2]
    neg_h, neg_t, neg_r = x[:, 3], x[:, 4], x[:, 5]
    h_ids = jnp.stack([pos_h, neg_h], axis=1).reshape(-1)
    t_ids = jnp.stack([pos_t, neg_t], axis=1).reshape(-1)
    r_ids = jnp.stack([pos_r, neg_r], axis=1).reshape(-1)

    mesh = plsc.VectorSubcoreMesh(core_axis_name="c", subcore_axis_name="s")
    run = functools.partial(
        pl.kernel,
        out_type=jax.ShapeDtypeStruct((_NW, 16), jnp.float32),
        mesh=mesh,
        scratch_types=[
            pltpu.VMEM((_JPW,), jnp.int32),
            pltpu.VMEM((_JPW,), jnp.int32),
            pltpu.VMEM((_JPW,), jnp.int32),
            pltpu.VMEM((_CH, D * D), jnp.float32),
            pltpu.VMEM((_CH, D), jnp.float32),
            pltpu.VMEM((_CH, D), jnp.float32),
            pltpu.VMEM((_CH, D), jnp.float32),
            pltpu.VMEM((16,), jnp.float32),
            pltpu.SemaphoreType.DMA,
        ],
        compiler_params=pltpu.CompilerParams(use_tc_tiling_on_sc=False),
    )(_sc_body)
    return run(h_ids, t_ids, r_ids, ent_emb, rel_emb, transfer)


def kernel(x, ent_emb, rel_emb, transfer, l1_flag):
    part = _run_sc(x, ent_emb, rel_emb, transfer)
    loss1 = jnp.sum(part[:, 0])
    loss2 = jnp.sum(part[:, 1])
    return jnp.where(l1_flag, loss1, loss2)


# speedup vs baseline: 1.9637x; 1.9637x over previous
import functools

import jax
import jax.numpy as jnp
from jax import lax
from jax.experimental import pallas as pl
from jax.experimental.pallas import tpu as pltpu
from jax.experimental.pallas import tpu_sc as plsc

D = 64
B = 4096
MARGIN = 1.0

_NC = 2
_NS = 16
_NW = _NC * _NS
_JOBS = 2 * B
_JPW = _JOBS // _NW
_CH = 16
_NCHUNK = _JPW // _CH

def _allsum16(v):
    iota = lax.iota(jnp.int32, 16)
    for sh in (1, 2, 4, 8):
        v = v + v.at[iota ^ sh].get(mode="promise_in_bounds")
    return v


def _rsqrt16(x):
    i = lax.bitcast_convert_type(x, jnp.int32)
    y = lax.bitcast_convert_type(jnp.int32(0x5F3759DF) - (i >> 1), jnp.float32)
    for _ in range(3):
        y = y * (1.5 - 0.5 * x * y * y)
    return y


def _sc_body(hid_hbm, tid_hbm, rid_hbm, ent_hbm, rel_hbm, tr_hbm, out_hbm,
             hid_v, tid_v, rid_v, m_v, h_v, t_v, r_v, acc_v, sem):
    wid = lax.axis_index("s") * _NC + lax.axis_index("c")
    base = wid * _JPW
    pltpu.sync_copy(hid_hbm.at[pl.ds(base, _JPW)], hid_v)
    pltpu.sync_copy(tid_hbm.at[pl.ds(base, _JPW)], tid_v)
    pltpu.sync_copy(rid_hbm.at[pl.ds(base, _JPW)], rid_v)

    splats = [jnp.full((16,), i, jnp.int32) for i in range(16)]

    def job_scores(j):
        ah = [jnp.zeros((16,), jnp.float32) for _ in range(4)]
        at = [jnp.zeros((16,), jnp.float32) for _ in range(4)]
        for d16 in range(4):
            hv = h_v[j, pl.ds(d16 * 16, 16)]
            tv = t_v[j, pl.ds(d16 * 16, 16)]
            for dd in range(16):
                he = hv.at[splats[dd]].get(mode="promise_in_bounds")
                te = tv.at[splats[dd]].get(mode="promise_in_bounds")
                off = (d16 * 16 + dd) * D
                for k in range(4):
                    m = m_v[j, pl.ds(off + 16 * k, 16)]
                    ah[k] = ah[k] + he * m
                    at[k] = at[k] + te * m
        ssh = ah[0] * ah[0] + ah[1] * ah[1] + ah[2] * ah[2] + ah[3] * ah[3]
        sst = at[0] * at[0] + at[1] * at[1] + at[2] * at[2] + at[3] * at[3]
        inv_h = _rsqrt16(jnp.maximum(_allsum16(ssh), 1e-12))
        inv_t = _rsqrt16(jnp.maximum(_allsum16(sst), 1e-12))

        s_abs = jnp.zeros((16,), jnp.float32)
        s_sq = jnp.zeros((16,), jnp.float32)
        for k in range(4):
            dk = inv_h * ah[k] + r_v[j, pl.ds(16 * k, 16)] - inv_t * at[k]
            s_abs = s_abs + jnp.abs(dk)
            s_sq = s_sq + dk * dk
        return _allsum16(s_abs), _allsum16(s_sq)

    def chunk_body(c, carry):
        acc1_o, acc2_o = carry
        cb = c * _CH
        hidx = hid_v[pl.ds(cb, _CH)]
        tidx = tid_v[pl.ds(cb, _CH)]
        ridx = rid_v[pl.ds(cb, _CH)]
        cps = [pltpu.async_copy(tr_hbm.at[ridx], m_v, sem)]
        for jj in range(_CH):
            cps.append(pltpu.async_copy(ent_hbm.at[hidx[jj]], h_v.at[jj], sem))
            cps.append(pltpu.async_copy(ent_hbm.at[tidx[jj]], t_v.at[jj], sem))
            cps.append(pltpu.async_copy(rel_hbm.at[ridx[jj]], r_v.at[jj], sem))
        for cp in cps:
            cp.wait()

        def pair_body(p, pc):
            acc1, acc2 = pc
            p1, p2 = job_scores(2 * p)
            n1, n2 = job_scores(2 * p + 1)
            acc1 = acc1 + jnp.maximum(p1 - n1 + MARGIN, 0.0)
            acc2 = acc2 + jnp.maximum(p2 - n2 + MARGIN, 0.0)
            return acc1, acc2

        return lax.fori_loop(0, _CH // 2, pair_body, (acc1_o, acc2_o))

    zero16 = jnp.zeros((16,), jnp.float32)
    acc1, acc2 = lax.fori_loop(0, _NCHUNK, chunk_body, (zero16, zero16))

    lane = lax.iota(jnp.int32, 16)
    res = jnp.where(lane == 0, acc1, jnp.where(lane == 1, acc2, 0.0))
    acc_v[...] = res
    pltpu.sync_copy(acc_v, out_hbm.at[wid])


def _run_sc(x, ent_emb, rel_emb, transfer):
    pos_h, pos_t, pos_r = x[:, 0], x[:, 1], x[:, 2]
    neg_h, neg_t, neg_r = x[:, 3], x[:, 4], x[:, 5]
    h_ids = jnp.stack([pos_h, neg_h], axis=1).reshape(-1)
    t_ids = jnp.stack([pos_t, neg_t], axis=1).reshape(-1)
    r_ids = jnp.stack([pos_r, neg_r], axis=1).reshape(-1)

    mesh = plsc.VectorSubcoreMesh(core_axis_name="c", subcore_axis_name="s")
    run = functools.partial(
        pl.kernel,
        out_type=jax.ShapeDtypeStruct((_NW, 16), jnp.float32),
        mesh=mesh,
        scratch_types=[
            pltpu.VMEM((_JPW,), jnp.int32),
            pltpu.VMEM((_JPW,), jnp.int32),
            pltpu.VMEM((_JPW,), jnp.int32),
            pltpu.VMEM((_CH, D * D), jnp.float32),
            pltpu.VMEM((_CH, D), jnp.float32),
            pltpu.VMEM((_CH, D), jnp.float32),
            pltpu.VMEM((_CH, D), jnp.float32),
            pltpu.VMEM((16,), jnp.float32),
            pltpu.SemaphoreType.DMA,
        ],
    )(_sc_body)
    return run(h_ids, t_ids, r_ids, ent_emb, rel_emb, transfer)


def kernel(x, ent_emb, rel_emb, transfer, l1_flag):
    part = _run_sc(x, ent_emb, rel_emb, transfer)
    loss1 = jnp.sum(part[:, 0])
    loss2 = jnp.sum(part[:, 1])
    return jnp.where(l1_flag, loss1, loss2)
